# baseline (speedup 1.0000x reference)
import functools

import jax
import jax.numpy as jnp
from jax.experimental import pallas as pl
from jax.experimental.pallas import tpu as pltpu

_EPS = 1e-10


def _mlp_in(x, W1, b1, W2, b2):
    h = jax.nn.relu(
        jax.lax.dot_general(x, W1, (((1,), (0,)), ((), ())),
                            preferred_element_type=jnp.float32) + b1)
    return jax.nn.relu(
        jax.lax.dot_general(h, W2, (((1,), (0,)), ((), ())),
                            preferred_element_type=jnp.float32) + b2)


def _fused_kernel(a_ref, w_ref, x_ref,
                  w1n_ref, b1n_ref, w2n_ref, b2n_ref,
                  w1s_ref, b1s_ref, w2s_ref, b2s_ref,
                  w_out_ref, x2_ref, *, block_i):
    i = pl.program_id(1)
    x_full = x_ref[0]
    x1 = _mlp_in(x_full, w1n_ref[...], b1n_ref[...], w2n_ref[...], b2n_ref[...])

    a = a_ref[0]
    a_sum = jnp.sum(a, axis=1, keepdims=True) + _EPS
    an = a / a_sum

    w = w_ref[0]
    w_out_ref[0] = w

    msg = jnp.sum(w * x1[None, :, :] * an[:, :, None], axis=1)

    x_rows = x_ref[0, pl.ds(i * block_i, block_i), :]
    xs = _mlp_in(x_rows, w1s_ref[...], b1s_ref[...], w2s_ref[...], b2s_ref[...])
    x2_ref[0] = msg + xs


def kernel(A, W, x, W1_n, b1_n, W2_n, b2_n, W1_s, b1_s, W2_s, b2_s):
    b, n, _, f = W.shape
    block_i = 8

    b1n = b1_n.reshape(1, f)
    b2n = b2_n.reshape(1, f)
    b1s = b1_s.reshape(1, f)
    b2s = b2_s.reshape(1, f)

    grid = (b, n // block_i)
    small = lambda bi, ii: (0, 0)

    w_out, x2 = pl.pallas_call(
        functools.partial(_fused_kernel, block_i=block_i),
        grid=grid,
        in_specs=[
            pl.BlockSpec((1, block_i, n), lambda bi, ii: (bi, ii, 0)),
            pl.BlockSpec((1, block_i, n, f), lambda bi, ii: (bi, ii, 0, 0)),
            pl.BlockSpec((1, n, f), lambda bi, ii: (bi, 0, 0)),
            pl.BlockSpec((f, f), small),
            pl.BlockSpec((1, f), small),
            pl.BlockSpec((f, f), small),
            pl.BlockSpec((1, f), small),
            pl.BlockSpec((f, f), small),
            pl.BlockSpec((1, f), small),
            pl.BlockSpec((f, f), small),
            pl.BlockSpec((1, f), small),
        ],
        out_specs=[
            pl.BlockSpec((1, block_i, n, f), lambda bi, ii: (bi, ii, 0, 0)),
            pl.BlockSpec((1, block_i, f), lambda bi, ii: (bi, ii, 0)),
        ],
        out_shape=[
            jax.ShapeDtypeStruct(W.shape, W.dtype),
            jax.ShapeDtypeStruct((b, n, f), x.dtype),
        ],
        compiler_params=pltpu.CompilerParams(
            dimension_semantics=("parallel", "arbitrary"),
        ),
    )(A, W, x, W1_n, b1n, W2_n, b2n, W1_s, b1s, W2_s, b2s)
    return (w_out, x2)


# speedup vs baseline: 2.6734x; 2.6734x over previous
import functools

import jax
import jax.numpy as jnp
from jax.experimental import pallas as pl
from jax.experimental.pallas import tpu as pltpu

_EPS = 1e-10


def _packed_mlp(xp, W1p, b1p, W2p, b2p):
    h = jax.nn.relu(
        jax.lax.dot_general(xp, W1p, (((1,), (0,)), ((), ())),
                            preferred_element_type=jnp.float32) + b1p)
    return jax.nn.relu(
        jax.lax.dot_general(h, W2p, (((1,), (0,)), ((), ())),
                            preferred_element_type=jnp.float32) + b2p)


def _fused_kernel(a_ref, w_ref, xp_ref, xu_ref,
                  w1np_ref, b1np_ref, w2np_ref, b2np_ref,
                  w1s_ref, b1s_ref, w2s_ref, b2s_ref,
                  w_out_ref, x2_ref, *, block_i):
    f32 = jnp.float32

    xp = xp_ref[0]
    x1p = _packed_mlp(xp, w1np_ref[...], b1np_ref[...],
                      w2np_ref[...], b2np_ref[...])

    a4 = a_ref[0]
    a_sum = jnp.sum(jnp.sum(a4, axis=2), axis=1, keepdims=True) + _EPS

    a3 = jnp.reshape(a4, (block_i * 128, 8))
    rep = (jax.lax.broadcasted_iota(jnp.int32, (8, 128), 1) // 16
           == jax.lax.broadcasted_iota(jnp.int32, (8, 128), 0)).astype(f32)
    a_exp = jax.lax.dot_general(a3, rep, (((1,), (0,)), ((), ())),
                                preferred_element_type=f32)
    a_exp = jnp.reshape(a_exp, (block_i, 128, 128))

    w = w_ref[0]
    w_out_ref[0] = w

    q = w * x1p[None, :, :] * a_exp
    s = jnp.sum(q, axis=1)

    sel = (jax.lax.broadcasted_iota(jnp.int32, (128, 16), 0) % 16
           == jax.lax.broadcasted_iota(jnp.int32, (128, 16), 1)).astype(f32)
    x_new = jax.lax.dot_general(s, sel, (((1,), (0,)), ((), ())),
                                preferred_element_type=f32) / a_sum

    xu = xu_ref[0]
    h = jax.nn.relu(
        jax.lax.dot_general(xu, w1s_ref[...], (((1,), (0,)), ((), ())),
                            preferred_element_type=f32) + b1s_ref[...])
    xs = jax.nn.relu(
        jax.lax.dot_general(h, w2s_ref[...], (((1,), (0,)), ((), ())),
                            preferred_element_type=f32) + b2s_ref[...])

    x2_ref[0] = x_new + xs


def kernel(A, W, x, W1_n, b1_n, W2_n, b2_n, W1_s, b1_s, W2_s, b2_s):
    b, n, _, f = W.shape
    block_i = 64

    W4 = W.reshape(b, n, 128, 128)
    A4 = A.reshape(b, n, 128, 8)
    xp = x.reshape(b, 128, 128)

    ii = jnp.arange(128) // f
    bd_mask = (ii[:, None] == ii[None, :]).astype(W1_n.dtype)
    W1np = jnp.tile(W1_n, (8, 8)) * bd_mask
    W2np = jnp.tile(W2_n, (8, 8)) * bd_mask
    b1np = jnp.tile(b1_n, (8,)).reshape(1, 128)
    b2np = jnp.tile(b2_n, (8,)).reshape(1, 128)
    b1s = b1_s.reshape(1, f)
    b2s = b2_s.reshape(1, f)

    grid = (b, n // block_i)
    small = lambda bi, ii_: (0, 0)

    w_out, x2 = pl.pallas_call(
        functools.partial(_fused_kernel, block_i=block_i),
        grid=grid,
        in_specs=[
            pl.BlockSpec((1, block_i, 128, 8), lambda bi, ii_: (bi, ii_, 0, 0)),
            pl.BlockSpec((1, block_i, 128, 128), lambda bi, ii_: (bi, ii_, 0, 0)),
            pl.BlockSpec((1, 128, 128), lambda bi, ii_: (bi, 0, 0)),
            pl.BlockSpec((1, block_i, f), lambda bi, ii_: (bi, ii_, 0)),
            pl.BlockSpec((128, 128), small),
            pl.BlockSpec((1, 128), small),
            pl.BlockSpec((128, 128), small),
            pl.BlockSpec((1, 128), small),
            pl.BlockSpec((f, f), small),
            pl.BlockSpec((1, f), small),
            pl.BlockSpec((f, f), small),
            pl.BlockSpec((1, f), small),
        ],
        out_specs=[
            pl.BlockSpec((1, block_i, 128, 128), lambda bi, ii_: (bi, ii_, 0, 0)),
            pl.BlockSpec((1, block_i, f), lambda bi, ii_: (bi, ii_, 0)),
        ],
        out_shape=[
            jax.ShapeDtypeStruct((b, n, 128, 128), W.dtype),
            jax.ShapeDtypeStruct((b, n, f), x.dtype),
        ],
        compiler_params=pltpu.CompilerParams(
            dimension_semantics=("parallel", "parallel"),
        ),
    )(A4, W4, xp, x, W1np, b1np, W2np, b2np, W1_s, b1s, W2_s, b2s)
    return (w_out.reshape(b, n, n, f), x2)


# speedup vs baseline: 17.6996x; 6.6207x over previous
import functools

import jax
import jax.numpy as jnp
from jax.experimental import pallas as pl
from jax.experimental.pallas import tpu as pltpu

_EPS = 1e-10


def _mlp_t(xt, W1t, b1, W2t, b2):
    h = jax.nn.relu(
        jax.lax.dot_general(W1t, xt, (((1,), (0,)), ((), ())),
                            preferred_element_type=jnp.float32) + b1)
    return jax.nn.relu(
        jax.lax.dot_general(W2t, h, (((1,), (0,)), ((), ())),
                            preferred_element_type=jnp.float32) + b2)


def _fused_kernel(a_ref, w_ref, xt_ref, xb_ref,
                  w1nt_ref, b1n_ref, w2nt_ref, b2n_ref,
                  w1st_ref, b1s_ref, w2st_ref, b2s_ref,
                  w_out_ref, x2_ref):
    xt = xt_ref[0]
    x1t = _mlp_t(xt, w1nt_ref[...], b1n_ref[...], w2nt_ref[...], b2n_ref[...])

    a = a_ref[0]
    a_sum = jnp.sum(a, axis=1, keepdims=True) + _EPS

    w = w_ref[0]
    w_out_ref[0] = w

    q = w * x1t[None, :, :] * a[:, None, :]
    m = jnp.sum(q, axis=2) / a_sum

    xs_t = _mlp_t(xb_ref[0], w1st_ref[...], b1s_ref[...],
                  w2st_ref[...], b2s_ref[...])
    x2_ref[0] = xs_t + m.T


def kernel(A, W, x, W1_n, b1_n, W2_n, b2_n, W1_s, b1_s, W2_s, b2_s):
    b, n, _, f = W.shape
    block_i = 128

    Wt = jnp.transpose(W, (0, 1, 3, 2))
    xt = jnp.transpose(x, (0, 2, 1))

    w1nt = W1_n.T
    w2nt = W2_n.T
    w1st = W1_s.T
    w2st = W2_s.T
    b1n = b1_n.reshape(f, 1)
    b2n = b2_n.reshape(f, 1)
    b1s = b1_s.reshape(f, 1)
    b2s = b2_s.reshape(f, 1)

    grid = (b, n // block_i)
    small = lambda bi, ii: (0, 0)

    w_out, x2t = pl.pallas_call(
        _fused_kernel,
        grid=grid,
        in_specs=[
            pl.BlockSpec((1, block_i, n), lambda bi, ii: (bi, ii, 0)),
            pl.BlockSpec((1, block_i, f, n), lambda bi, ii: (bi, ii, 0, 0)),
            pl.BlockSpec((1, f, n), lambda bi, ii: (bi, 0, 0)),
            pl.BlockSpec((1, f, block_i), lambda bi, ii: (bi, 0, ii)),
            pl.BlockSpec((f, f), small),
            pl.BlockSpec((f, 1), small),
            pl.BlockSpec((f, f), small),
            pl.BlockSpec((f, 1), small),
            pl.BlockSpec((f, f), small),
            pl.BlockSpec((f, 1), small),
            pl.BlockSpec((f, f), small),
            pl.BlockSpec((f, 1), small),
        ],
        out_specs=[
            pl.BlockSpec((1, block_i, f, n), lambda bi, ii: (bi, ii, 0, 0)),
            pl.BlockSpec((1, f, block_i), lambda bi, ii: (bi, 0, ii)),
        ],
        out_shape=[
            jax.ShapeDtypeStruct((b, n, f, n), W.dtype),
            jax.ShapeDtypeStruct((b, f, n), x.dtype),
        ],
        compiler_params=pltpu.CompilerParams(
            dimension_semantics=("parallel", "parallel"),
        ),
    )(A, Wt, xt, xt, w1nt, b1n, w2nt, b2n, w1st, b1s, w2st, b2s)
    return (jnp.transpose(w_out, (0, 1, 3, 2)), jnp.transpose(x2t, (0, 2, 1)))


# speedup vs baseline: 18.6718x; 1.0549x over previous
import functools

import jax
import jax.numpy as jnp
from jax.experimental import pallas as pl
from jax.experimental.pallas import tpu as pltpu

_EPS = 1e-10


def _mlp_t(xt, W1, b1, W2, b2):
    h = jax.nn.relu(
        jax.lax.dot_general(W1, xt, (((0,), (0,)), ((), ())),
                            preferred_element_type=jnp.float32) + b1)
    return jax.nn.relu(
        jax.lax.dot_general(W2, h, (((0,), (0,)), ((), ())),
                            preferred_element_type=jnp.float32) + b2)


def _fused_kernel(a_ref, w_ref, xt_ref, xb_ref,
                  w1nt_ref, b1n_ref, w2nt_ref, b2n_ref,
                  w1st_ref, b1s_ref, w2st_ref, b2s_ref,
                  w_out_ref, x2_ref):
    xt = xt_ref[0]
    x1t = _mlp_t(xt, w1nt_ref[...], b1n_ref[...], w2nt_ref[...], b2n_ref[...])

    a = a_ref[0]
    a_sum = jnp.sum(a, axis=1, keepdims=True) + _EPS

    w = w_ref[0]
    w_out_ref[0] = w

    q = w * x1t[None, :, :] * a[:, None, :]
    m = jnp.sum(q, axis=2) / a_sum

    xs_t = _mlp_t(xb_ref[0], w1st_ref[...], b1s_ref[...],
                  w2st_ref[...], b2s_ref[...])
    x2_ref[0] = xs_t + m.T


def kernel(A, W, x, W1_n, b1_n, W2_n, b2_n, W1_s, b1_s, W2_s, b2_s):
    b, n, _, f = W.shape
    block_i = 128

    Wt = jnp.transpose(W, (0, 1, 3, 2))
    xt = jnp.transpose(x, (0, 2, 1))

    b1n = b1_n.reshape(f, 1)
    b2n = b2_n.reshape(f, 1)
    b1s = b1_s.reshape(f, 1)
    b2s = b2_s.reshape(f, 1)

    grid = (b, n // block_i)
    small = lambda bi, ii: (0, 0)

    w_out, x2t = pl.pallas_call(
        _fused_kernel,
        grid=grid,
        in_specs=[
            pl.BlockSpec((1, block_i, n), lambda bi, ii: (bi, ii, 0)),
            pl.BlockSpec((1, block_i, f, n), lambda bi, ii: (bi, ii, 0, 0)),
            pl.BlockSpec((1, f, n), lambda bi, ii: (bi, 0, 0)),
            pl.BlockSpec((1, f, block_i), lambda bi, ii: (bi, 0, ii)),
            pl.BlockSpec((f, f), small),
            pl.BlockSpec((f, 1), small),
            pl.BlockSpec((f, f), small),
            pl.BlockSpec((f, 1), small),
            pl.BlockSpec((f, f), small),
            pl.BlockSpec((f, 1), small),
            pl.BlockSpec((f, f), small),
            pl.BlockSpec((f, 1), small),
        ],
        out_specs=[
            pl.BlockSpec((1, block_i, f, n), lambda bi, ii: (bi, ii, 0, 0)),
            pl.BlockSpec((1, f, block_i), lambda bi, ii: (bi, 0, ii)),
        ],
        out_shape=[
            jax.ShapeDtypeStruct((b, n, f, n), W.dtype),
            jax.ShapeDtypeStruct((b, f, n), x.dtype),
        ],
        compiler_params=pltpu.CompilerParams(
            dimension_semantics=("parallel", "parallel"),
        ),
    )(A, Wt, xt, xt, W1_n, b1n, W2_n, b2n, W1_s, b1s, W2_s, b2s)
    return (jnp.transpose(w_out, (0, 1, 3, 2)), jnp.transpose(x2t, (0, 2, 1)))


# speedup vs baseline: 18.9555x; 1.0152x over previous
import jax
import jax.numpy as jnp
from jax.experimental import pallas as pl
from jax.experimental.pallas import tpu as pltpu

_EPS = 1e-10

_BI = 64
_NBUF = 6
_DEPTH = 3


def _mlp_t(xt, W1, b1, W2, b2):
    h = jax.nn.relu(
        jax.lax.dot_general(W1, xt, (((0,), (0,)), ((), ())),
                            preferred_element_type=jnp.float32) + b1)
    return jax.nn.relu(
        jax.lax.dot_general(W2, h, (((0,), (0,)), ((), ())),
                            preferred_element_type=jnp.float32) + b2)


def _make_body(b, n, f):
    n_blk = n // _BI
    n_step = b * n_blk

    def body(a_hbm, w_hbm, xt_vmem,
             w1n, b1n, w2n, b2n, w1s, b1s, w2s, b2s,
             w_out_hbm, x2_vmem,
             wbuf, abuf, x1_vmem, xs_vmem, in_sem, a_in_sem, out_sem):
        for bi in range(b):
            xt = xt_vmem[bi]
            x1_vmem[bi] = _mlp_t(xt, w1n[...], b1n[...], w2n[...], b2n[...])
            xs_vmem[bi] = _mlp_t(xt, w1s[...], b1s[...], w2s[...], b2s[...])

        def start_in(t):
            slot = t % _NBUF
            bi, blk = divmod(t, n_blk)
            i0 = blk * _BI
            pltpu.make_async_copy(w_hbm.at[bi, pl.ds(i0, _BI)],
                                  wbuf.at[slot], in_sem.at[slot]).start()
            pltpu.make_async_copy(a_hbm.at[bi, pl.ds(i0, _BI)],
                                  abuf.at[slot], a_in_sem.at[slot]).start()

        def out_copy(t):
            slot = t % _NBUF
            bi, blk = divmod(t, n_blk)
            i0 = blk * _BI
            return pltpu.make_async_copy(wbuf.at[slot],
                                         w_out_hbm.at[bi, pl.ds(i0, _BI)],
                                         out_sem.at[slot])

        for t in range(min(_DEPTH, n_step)):
            start_in(t)

        for t in range(n_step):
            slot = t % _NBUF
            bi, blk = divmod(t, n_blk)
            i0 = blk * _BI

            pltpu.make_async_copy(w_hbm.at[bi, pl.ds(i0, _BI)],
                                  wbuf.at[slot], in_sem.at[slot]).wait()
            pltpu.make_async_copy(a_hbm.at[bi, pl.ds(i0, _BI)],
                                  abuf.at[slot], a_in_sem.at[slot]).wait()

            a = abuf[slot]
            a_sum = jnp.sum(a, axis=1, keepdims=True) + _EPS
            w = wbuf[slot]
            q = w * x1_vmem[bi][None, :, :] * a[:, None, :]
            m = jnp.sum(q, axis=2) / a_sum
            x2_vmem[bi, :, i0:i0 + _BI] = xs_vmem[bi, :, i0:i0 + _BI] + m.T

            out_copy(t).start()

            nxt = t + _DEPTH
            if nxt < n_step:
                prev = nxt - _NBUF
                if prev >= 0:
                    out_copy(prev).wait()
                start_in(nxt)

        for t in range(max(0, n_step - _NBUF), n_step):
            out_copy(t).wait()

    return body


def kernel(A, W, x, W1_n, b1_n, W2_n, b2_n, W1_s, b1_s, W2_s, b2_s):
    b, n, _, f = W.shape

    Wt = jnp.transpose(W, (0, 1, 3, 2))
    xt = jnp.transpose(x, (0, 2, 1))

    b1n = b1_n.reshape(f, 1)
    b2n = b2_n.reshape(f, 1)
    b1s = b1_s.reshape(f, 1)
    b2s = b2_s.reshape(f, 1)

    vsmall = pl.BlockSpec(memory_space=pltpu.VMEM)
    any_ = pl.BlockSpec(memory_space=pl.ANY)

    w_out, x2t = pl.pallas_call(
        _make_body(b, n, f),
        in_specs=[any_, any_, vsmall,
                  vsmall, vsmall, vsmall, vsmall,
                  vsmall, vsmall, vsmall, vsmall],
        out_specs=[any_, vsmall],
        out_shape=[
            jax.ShapeDtypeStruct((b, n, f, n), W.dtype),
            jax.ShapeDtypeStruct((b, f, n), x.dtype),
        ],
        scratch_shapes=[
            pltpu.VMEM((_NBUF, _BI, f, n), jnp.float32),
            pltpu.VMEM((_NBUF, _BI, n), jnp.float32),
            pltpu.VMEM((b, f, n), jnp.float32),
            pltpu.VMEM((b, f, n), jnp.float32),
            pltpu.SemaphoreType.DMA((_NBUF,)),
            pltpu.SemaphoreType.DMA((_NBUF,)),
            pltpu.SemaphoreType.DMA((_NBUF,)),
        ],
    )(A, Wt, xt, W1_n, b1n, W2_n, b2n, W1_s, b1s, W2_s, b2s)
    return (jnp.transpose(w_out, (0, 1, 3, 2)), jnp.transpose(x2t, (0, 2, 1)))


# speedup vs baseline: 18.9755x; 1.0011x over previous
import jax
import jax.numpy as jnp
from jax.experimental import pallas as pl
from jax.experimental.pallas import tpu as pltpu

_EPS = 1e-10

_BI = 64
_NBUF = 8
_DEPTH = 4


def _mlp_t(xt, W1, b1, W2, b2):
    h = jax.nn.relu(
        jax.lax.dot_general(W1, xt, (((0,), (0,)), ((), ())),
                            preferred_element_type=jnp.float32) + b1)
    return jax.nn.relu(
        jax.lax.dot_general(W2, h, (((0,), (0,)), ((), ())),
                            preferred_element_type=jnp.float32) + b2)


def _make_body(b, n, f):
    n_blk = n // _BI
    n_step = b * n_blk

    def body(a_hbm, w_hbm, xt_vmem,
             w1n, b1n, w2n, b2n, w1s, b1s, w2s, b2s,
             w_out_hbm, x2_vmem,
             wbuf, abuf, x1_vmem, xs_vmem, in_sem, a_in_sem, out_sem):
        for bi in range(b):
            xt = xt_vmem[bi]
            x1_vmem[bi] = _mlp_t(xt, w1n[...], b1n[...], w2n[...], b2n[...])
            xs_vmem[bi] = _mlp_t(xt, w1s[...], b1s[...], w2s[...], b2s[...])

        def start_in(t):
            slot = t % _NBUF
            bi, blk = divmod(t, n_blk)
            i0 = blk * _BI
            pltpu.make_async_copy(w_hbm.at[bi, pl.ds(i0, _BI)],
                                  wbuf.at[slot], in_sem.at[slot]).start()
            pltpu.make_async_copy(a_hbm.at[bi, pl.ds(i0, _BI)],
                                  abuf.at[slot], a_in_sem.at[slot]).start()

        def out_copy(t):
            slot = t % _NBUF
            bi, blk = divmod(t, n_blk)
            i0 = blk * _BI
            return pltpu.make_async_copy(wbuf.at[slot],
                                         w_out_hbm.at[bi, pl.ds(i0, _BI)],
                                         out_sem.at[slot])

        for t in range(min(_DEPTH, n_step)):
            start_in(t)

        for t in range(n_step):
            slot = t % _NBUF
            bi, blk = divmod(t, n_blk)
            i0 = blk * _BI

            pltpu.make_async_copy(w_hbm.at[bi, pl.ds(i0, _BI)],
                                  wbuf.at[slot], in_sem.at[slot]).wait()
            pltpu.make_async_copy(a_hbm.at[bi, pl.ds(i0, _BI)],
                                  abuf.at[slot], a_in_sem.at[slot]).wait()

            a = abuf[slot]
            a_sum = jnp.sum(a, axis=1, keepdims=True) + _EPS
            w = wbuf[slot]
            q = w * x1_vmem[bi][None, :, :] * a[:, None, :]
            m = jnp.sum(q, axis=2) / a_sum
            x2_vmem[bi, :, i0:i0 + _BI] = xs_vmem[bi, :, i0:i0 + _BI] + m.T

            out_copy(t).start()

            nxt = t + _DEPTH
            if nxt < n_step:
                prev = nxt - _NBUF
                if prev >= 0:
                    out_copy(prev).wait()
                start_in(nxt)

        for t in range(max(0, n_step - _NBUF), n_step):
            out_copy(t).wait()

    return body


def kernel(A, W, x, W1_n, b1_n, W2_n, b2_n, W1_s, b1_s, W2_s, b2_s):
    b, n, _, f = W.shape

    Wt = jnp.transpose(W, (0, 1, 3, 2))
    xt = jnp.transpose(x, (0, 2, 1))

    b1n = b1_n.reshape(f, 1)
    b2n = b2_n.reshape(f, 1)
    b1s = b1_s.reshape(f, 1)
    b2s = b2_s.reshape(f, 1)

    vsmall = pl.BlockSpec(memory_space=pltpu.VMEM)
    any_ = pl.BlockSpec(memory_space=pl.ANY)

    w_out, x2t = pl.pallas_call(
        _make_body(b, n, f),
        in_specs=[any_, any_, vsmall,
                  vsmall, vsmall, vsmall, vsmall,
                  vsmall, vsmall, vsmall, vsmall],
        out_specs=[any_, vsmall],
        out_shape=[
            jax.ShapeDtypeStruct((b, n, f, n), W.dtype),
            jax.ShapeDtypeStruct((b, f, n), x.dtype),
        ],
        scratch_shapes=[
            pltpu.VMEM((_NBUF, _BI, f, n), jnp.float32),
            pltpu.VMEM((_NBUF, _BI, n), jnp.float32),
            pltpu.VMEM((b, f, n), jnp.float32),
            pltpu.VMEM((b, f, n), jnp.float32),
            pltpu.SemaphoreType.DMA((_NBUF,)),
            pltpu.SemaphoreType.DMA((_NBUF,)),
            pltpu.SemaphoreType.DMA((_NBUF,)),
        ],
    )(A, Wt, xt, W1_n, b1n, W2_n, b2n, W1_s, b1s, W2_s, b2s)
    return (jnp.transpose(w_out, (0, 1, 3, 2)), jnp.transpose(x2t, (0, 2, 1)))
